# Initial kernel scaffold; baseline (speedup 1.0000x reference)
import jax
import jax.numpy as jnp
from jax.experimental import pallas as pl


def kernel(meter, length, remainder, meter_table, leng_table, rem_table, W, b):
    raise NotImplementedError("write your pallas kernel here")


# speedup vs baseline: 6.3776x; 6.3776x over previous
import functools

import jax
import jax.numpy as jnp
from jax import lax
from jax.experimental import pallas as pl
from jax.experimental.pallas import tpu as pltpu
from jax.experimental.pallas import tpu_sc as plsc

B, L, D = 4096, 200, 64
N_TOK = B * L
METER_VOCAB, LENG_VOCAB, REM_VOCAB = 100000, 1024, 1024

NC, NS, LANES = 2, 16, 16
NW = NC * NS
GW = 128
CHUNK = 512
ROWS_PER_CHUNK = CHUNK // GW
TOK_PER_W = N_TOK // NW
ROWS_PER_W = TOK_PER_W // GW
N_CHUNKS = TOK_PER_W // CHUNK

MXU_BLK = 5000


def _proj_block_kernel(t_ref, w_ref, o_ref):
    o_ref[...] = jnp.dot(t_ref[...], w_ref[...],
                         preferred_element_type=jnp.float32,
                         precision=lax.Precision.HIGHEST)


def _proj_bias_kernel(t_ref, w_ref, b_ref, o_ref):
    o_ref[...] = jnp.dot(t_ref[...], w_ref[...],
                         preferred_element_type=jnp.float32,
                         precision=lax.Precision.HIGHEST) + b_ref[...]


def _project_meter(table, w):
    return pl.pallas_call(
        _proj_block_kernel,
        grid=(METER_VOCAB // MXU_BLK,),
        in_specs=[pl.BlockSpec((MXU_BLK, D), lambda i: (i, 0)),
                  pl.BlockSpec((D, D), lambda i: (0, 0))],
        out_specs=pl.BlockSpec((MXU_BLK, D), lambda i: (i, 0)),
        out_shape=jax.ShapeDtypeStruct((METER_VOCAB, D), jnp.float32),
    )(table, w)


def _project_small(table, w, bias=None):
    if bias is None:
        return pl.pallas_call(
            _proj_block_kernel,
            out_shape=jax.ShapeDtypeStruct(table.shape, jnp.float32),
        )(table, w)
    return pl.pallas_call(
        _proj_bias_kernel,
        out_shape=jax.ShapeDtypeStruct(table.shape, jnp.float32),
    )(table, w, bias)


_mesh = plsc.VectorSubcoreMesh(core_axis_name="c", subcore_axis_name="s")


@functools.partial(
    pl.kernel,
    out_type=jax.ShapeDtypeStruct((N_TOK, D), jnp.float32),
    mesh=_mesh,
    compiler_params=pltpu.CompilerParams(use_tc_tiling_on_sc=False),
    scratch_types=[
        pltpu.VMEM((ROWS_PER_CHUNK, GW), jnp.int32),
        pltpu.VMEM((ROWS_PER_CHUNK, GW), jnp.int32),
        pltpu.VMEM((ROWS_PER_CHUNK, GW), jnp.int32),
        pltpu.VMEM((CHUNK, D), jnp.float32),
        pltpu.VMEM((CHUNK, D), jnp.float32),
        pltpu.VMEM((CHUNK, D), jnp.float32),
        pltpu.SemaphoreType.DMA,
    ],
)
def _sc_gather_sum(pm_hbm, plb_hbm, pr_hbm, im_hbm, il_hbm, ir_hbm, out_hbm,
                   im_v, il_v, ir_v, am_v, al_v, ar_v, sem):
    wid = lax.axis_index("s") * NC + lax.axis_index("c")

    @pl.loop(0, N_CHUNKS)
    def _chunk(ci):
        tok_off = wid * TOK_PER_W + ci * CHUNK
        row_off = wid * ROWS_PER_W + ci * ROWS_PER_CHUNK
        pltpu.sync_copy(im_hbm.at[pl.ds(row_off, ROWS_PER_CHUNK)], im_v)
        pltpu.sync_copy(il_hbm.at[pl.ds(row_off, ROWS_PER_CHUNK)], il_v)
        pltpu.sync_copy(ir_hbm.at[pl.ds(row_off, ROWS_PER_CHUNK)], ir_v)
        copies = []
        for k in range(ROWS_PER_CHUNK):
            dst = pl.ds(k * GW, GW)
            copies.append(pltpu.async_copy(pm_hbm.at[im_v.at[k]],
                                           am_v.at[dst], sem))
            copies.append(pltpu.async_copy(plb_hbm.at[il_v.at[k]],
                                           al_v.at[dst], sem))
            copies.append(pltpu.async_copy(pr_hbm.at[ir_v.at[k]],
                                           ar_v.at[dst], sem))
        for c in copies:
            c.wait()

        @pl.loop(0, CHUNK)
        def _row(r):
            for j in range(D // LANES):
                sl = (r, pl.ds(j * LANES, LANES))
                am_v[sl] = am_v[sl] + al_v[sl] + ar_v[sl]

        pltpu.sync_copy(am_v, out_hbm.at[pl.ds(tok_off, CHUNK)])


def kernel(meter, length, remainder, meter_table, leng_table, rem_table, W, b):
    pm = _project_meter(meter_table, W[:D])
    plb = _project_small(leng_table, W[D:2 * D], b.reshape(1, D))
    pr = _project_small(rem_table, W[2 * D:])
    im = meter.astype(jnp.int32).reshape(N_TOK // GW, GW)
    il = length.astype(jnp.int32).reshape(N_TOK // GW, GW)
    ir = remainder.astype(jnp.int32).reshape(N_TOK // GW, GW)
    out = _sc_gather_sum(pm, plb, pr, im, il, ir)
    return out.reshape(B, L, D)


# speedup vs baseline: 7.1056x; 1.1142x over previous
import functools

import jax
import jax.numpy as jnp
from jax import lax
from jax.experimental import pallas as pl
from jax.experimental.pallas import tpu as pltpu
from jax.experimental.pallas import tpu_sc as plsc

B, L, D = 4096, 200, 64
N_TOK = B * L
METER_VOCAB = 100000

NC, NS, LANES = 2, 16, 16
NW = NC * NS
GW = 128
CHUNK = 200
TOK_PER_W = N_TOK // NW
N_CHUNKS = TOK_PER_W // CHUNK
IDX_PER_CHUNK = 3 * CHUNK

MXU_BLK = 5000

_SPLITS = [(0, GW), (GW, CHUNK - GW)]
_GATHER_SLICES = [[(t * CHUNK + o, n) for (o, n) in _SPLITS] for t in range(3)]


def _proj_block_kernel(t_ref, w_ref, o_ref):
    o_ref[...] = jnp.dot(t_ref[...], w_ref[...],
                         preferred_element_type=jnp.float32,
                         precision=lax.Precision.HIGHEST)


def _proj_bias_kernel(t_ref, w_ref, b_ref, o_ref):
    o_ref[...] = jnp.dot(t_ref[...], w_ref[...],
                         preferred_element_type=jnp.float32,
                         precision=lax.Precision.HIGHEST) + b_ref[...]


def _project_meter(table, w):
    return pl.pallas_call(
        _proj_block_kernel,
        grid=(METER_VOCAB // MXU_BLK,),
        in_specs=[pl.BlockSpec((MXU_BLK, D), lambda i: (i, 0)),
                  pl.BlockSpec((D, D), lambda i: (0, 0))],
        out_specs=pl.BlockSpec((MXU_BLK, D), lambda i: (i, 0)),
        out_shape=jax.ShapeDtypeStruct((METER_VOCAB, D), jnp.float32),
    )(table, w)


def _project_small(table, w, bias=None):
    if bias is None:
        return pl.pallas_call(
            _proj_block_kernel,
            out_shape=jax.ShapeDtypeStruct(table.shape, jnp.float32),
        )(table, w)
    return pl.pallas_call(
        _proj_bias_kernel,
        out_shape=jax.ShapeDtypeStruct(table.shape, jnp.float32),
    )(table, w, bias)


_mesh = plsc.VectorSubcoreMesh(core_axis_name="c", subcore_axis_name="s")

_scratch = []
for _slot in range(2):
    _scratch += [pltpu.VMEM((IDX_PER_CHUNK,), jnp.int32)]
    _scratch += [pltpu.VMEM((CHUNK, D), jnp.float32)] * 4
_scratch += [pltpu.SemaphoreType.DMA] * 6


@functools.partial(
    pl.kernel,
    out_type=jax.ShapeDtypeStruct((N_TOK, D), jnp.float32),
    mesh=_mesh,
    compiler_params=pltpu.CompilerParams(use_tc_tiling_on_sc=False),
    scratch_types=_scratch,
)
def _sc_gather_sum(pm_hbm, plb_hbm, pr_hbm, idx_hbm, out_hbm,
                   idx0, bm0, bl0, br0, o0,
                   idx1, bm1, bl1, br1, o1,
                   semi0, semi1, semg0, semg1, semo0, semo1):
    wid = lax.axis_index("s") * NC + lax.axis_index("c")
    idx_v = (idx0, idx1)
    bufs = ((bm0, bl0, br0), (bm1, bl1, br1))
    o_v = (o0, o1)
    semi = (semi0, semi1)
    semg = (semg0, semg1)
    semo = (semo0, semo1)
    tables = (pm_hbm, plb_hbm, pr_hbm)
    idx_base = wid * N_CHUNKS * IDX_PER_CHUNK

    def load_idx(ci, p, sem_mode):
        src = idx_hbm.at[pl.ds(idx_base + ci * IDX_PER_CHUNK, IDX_PER_CHUNK)]
        if sem_mode is None:
            pltpu.sync_copy(src, idx_v[p])
        else:
            pltpu.async_copy(src, idx_v[p], semi[p])

    def fire_gathers(ci, p):
        copies = []
        for t in range(3):
            for (off, n) in _GATHER_SLICES[t]:
                copies.append(pltpu.async_copy(
                    tables[t].at[idx_v[p].at[pl.ds(off, n)]],
                    bufs[p][t].at[pl.ds(off - t * CHUNK, n)],
                    semg[p]))
        return copies

    def drain_gathers(p):
        for t in range(3):
            for (off, n) in _GATHER_SLICES[t]:
                pltpu.make_async_copy(
                    tables[t].at[idx_v[p].at[pl.ds(off, n)]],
                    bufs[p][t].at[pl.ds(off - t * CHUNK, n)],
                    semg[p]).wait()

    load_idx(0, 0, None)
    fire_gathers(0, 0)
    load_idx(1, 1, "async")

    @pl.loop(0, N_CHUNKS, step=2)
    def _pair(ci0):
        for p in range(2):
            q = 1 - p
            ci = ci0 + p
            bm, bl, br = bufs[p]

            @pl.when(ci + 1 < N_CHUNKS)
            def _():
                pltpu.make_async_copy(
                    idx_hbm.at[pl.ds(idx_base, IDX_PER_CHUNK)],
                    idx_v[q], semi[q]).wait()
                fire_gathers(ci + 1, q)

            drain_gathers(p)

            @pl.when(ci + 2 < N_CHUNKS)
            def _():
                load_idx(ci + 2, p, "async")

            @pl.when(ci >= 2)
            def _():
                pltpu.make_async_copy(
                    out_hbm.at[pl.ds(0, CHUNK)], o_v[p], semo[p]).wait()

            @pl.loop(0, CHUNK, step=2)
            def _row(r):
                for rr in range(2):
                    for j in range(D // LANES):
                        sl = (r + rr, pl.ds(j * LANES, LANES))
                        o_v[p][sl] = bm[sl] + bl[sl] + br[sl]

            tok_off = wid * TOK_PER_W + ci * CHUNK
            pltpu.async_copy(o_v[p], out_hbm.at[pl.ds(tok_off, CHUNK)],
                             semo[p])

    pltpu.make_async_copy(out_hbm.at[pl.ds(0, CHUNK)], o0, semo0).wait()
    pltpu.make_async_copy(out_hbm.at[pl.ds(0, CHUNK)], o1, semo1).wait()


def _pack_indices(meter, length, remainder):
    m = meter.astype(jnp.int32).reshape(NW, N_CHUNKS, CHUNK)
    l = length.astype(jnp.int32).reshape(NW, N_CHUNKS, CHUNK)
    r = remainder.astype(jnp.int32).reshape(NW, N_CHUNKS, CHUNK)
    return jnp.stack([m, l, r], axis=2).reshape(-1)


def kernel(meter, length, remainder, meter_table, leng_table, rem_table, W, b):
    pm = _project_meter(meter_table, W[:D])
    plb = _project_small(leng_table, W[D:2 * D], b.reshape(1, D))
    pr = _project_small(rem_table, W[2 * D:])
    idx = _pack_indices(meter, length, remainder)
    out = _sc_gather_sum(pm, plb, pr, idx)
    return out.reshape(B, L, D)


# speedup vs baseline: 7.3017x; 1.0276x over previous
import functools

import jax
import jax.numpy as jnp
from jax import lax
from jax.experimental import pallas as pl
from jax.experimental.pallas import tpu as pltpu
from jax.experimental.pallas import tpu_sc as plsc

B, L, D = 4096, 200, 64
METER_VOCAB = 100000

NC, NS, LANES = 2, 16, 16
NW = NC * NS
GW = 128
CHUNK = L
ROWS_PER_W = B // NW
_SPLITS = [(0, GW), (GW, CHUNK - GW)]

MXU_BLK = 5000


def _proj_block_kernel(t_ref, w_ref, o_ref):
    o_ref[...] = jnp.dot(t_ref[...], w_ref[...],
                         preferred_element_type=jnp.float32,
                         precision=lax.Precision.HIGHEST)


def _proj_bias_kernel(t_ref, w_ref, b_ref, o_ref):
    o_ref[...] = jnp.dot(t_ref[...], w_ref[...],
                         preferred_element_type=jnp.float32,
                         precision=lax.Precision.HIGHEST) + b_ref[...]


def _project_meter(table, w):
    return pl.pallas_call(
        _proj_block_kernel,
        grid=(METER_VOCAB // MXU_BLK,),
        in_specs=[pl.BlockSpec((MXU_BLK, D), lambda i: (i, 0)),
                  pl.BlockSpec((D, D), lambda i: (0, 0))],
        out_specs=pl.BlockSpec((MXU_BLK, D), lambda i: (i, 0)),
        out_shape=jax.ShapeDtypeStruct((METER_VOCAB, D), jnp.float32),
    )(table, w)


def _project_small(table, w, bias=None):
    if bias is None:
        return pl.pallas_call(
            _proj_block_kernel,
            out_shape=jax.ShapeDtypeStruct(table.shape, jnp.float32),
        )(table, w)
    return pl.pallas_call(
        _proj_bias_kernel,
        out_shape=jax.ShapeDtypeStruct(table.shape, jnp.float32),
    )(table, w, bias)


_mesh = plsc.VectorSubcoreMesh(core_axis_name="c", subcore_axis_name="s")

_scratch = []
for _slot in range(2):
    _scratch += [pltpu.VMEM((CHUNK,), jnp.int32)] * 3
    _scratch += [pltpu.VMEM((CHUNK, D), jnp.float32)] * 4
_scratch += [pltpu.SemaphoreType.DMA] * 6


@functools.partial(
    pl.kernel,
    out_type=jax.ShapeDtypeStruct((B, L, D), jnp.float32),
    mesh=_mesh,
    compiler_params=pltpu.CompilerParams(use_tc_tiling_on_sc=False),
    scratch_types=_scratch,
)
def _sc_gather_sum(pm_hbm, plb_hbm, pr_hbm, im_hbm, il_hbm, ir_hbm, out_hbm,
                   im0, il0, ir0, bm0, bl0, br0, o0,
                   im1, il1, ir1, bm1, bl1, br1, o1,
                   semi0, semi1, semg0, semg1, semo0, semo1):
    wid = lax.axis_index("s") * NC + lax.axis_index("c")
    idx_v = ((im0, il0, ir0), (im1, il1, ir1))
    bufs = ((bm0, bl0, br0), (bm1, bl1, br1))
    o_v = (o0, o1)
    semi = (semi0, semi1)
    semg = (semg0, semg1)
    semo = (semo0, semo1)
    idx_hbm = (im_hbm, il_hbm, ir_hbm)
    tables = (pm_hbm, plb_hbm, pr_hbm)
    row_base = wid * ROWS_PER_W

    def load_idx(b, p, is_async):
        for t in range(3):
            src = idx_hbm[t].at[b]
            if is_async:
                pltpu.async_copy(src, idx_v[p][t], semi[p])
            else:
                pltpu.sync_copy(src, idx_v[p][t])

    def wait_idx(p):
        for t in range(3):
            pltpu.make_async_copy(idx_hbm[t].at[0], idx_v[p][t],
                                  semi[p]).wait()

    def fire_gathers(p):
        for t in range(3):
            for (off, n) in _SPLITS:
                pltpu.async_copy(
                    tables[t].at[idx_v[p][t].at[pl.ds(off, n)]],
                    bufs[p][t].at[pl.ds(off, n)],
                    semg[p])

    def drain_gathers(p):
        for t in range(3):
            for (off, n) in _SPLITS:
                pltpu.make_async_copy(
                    tables[t].at[idx_v[p][t].at[pl.ds(off, n)]],
                    bufs[p][t].at[pl.ds(off, n)],
                    semg[p]).wait()

    load_idx(row_base, 0, False)
    fire_gathers(0)
    load_idx(row_base + 1, 1, True)

    @pl.loop(0, ROWS_PER_W, step=2)
    def _pair(ci0):
        for p in range(2):
            q = 1 - p
            ci = ci0 + p
            bm, bl, br = bufs[p]

            @pl.when(ci + 1 < ROWS_PER_W)
            def _():
                wait_idx(q)
                fire_gathers(q)

            drain_gathers(p)

            @pl.when(ci + 2 < ROWS_PER_W)
            def _():
                load_idx(row_base + ci + 2, p, True)

            @pl.when(ci >= 2)
            def _():
                pltpu.make_async_copy(out_hbm.at[0], o_v[p], semo[p]).wait()

            @pl.loop(0, CHUNK, step=2)
            def _row(r):
                for rr in range(2):
                    for j in range(D // LANES):
                        sl = (r + rr, pl.ds(j * LANES, LANES))
                        o_v[p][sl] = bm[sl] + bl[sl] + br[sl]

            pltpu.async_copy(o_v[p], out_hbm.at[row_base + ci], semo[p])

    pltpu.make_async_copy(out_hbm.at[0], o0, semo0).wait()
    pltpu.make_async_copy(out_hbm.at[0], o1, semo1).wait()


def kernel(meter, length, remainder, meter_table, leng_table, rem_table, W, b):
    pm = _project_meter(meter_table, W[:D])
    plb = _project_small(leng_table, W[D:2 * D], b.reshape(1, D))
    pr = _project_small(rem_table, W[2 * D:])
    im = meter.astype(jnp.int32)
    il = length.astype(jnp.int32)
    ir = remainder.astype(jnp.int32)
    return _sc_gather_sum(pm, plb, pr, im, il, ir)


# speedup vs baseline: 8.1317x; 1.1137x over previous
import functools

import jax
import jax.numpy as jnp
from jax import lax
from jax.experimental import pallas as pl
from jax.experimental.pallas import tpu as pltpu
from jax.experimental.pallas import tpu_sc as plsc

B, L, D = 4096, 200, 64
METER_VOCAB = 100000

NC, NS, LANES = 2, 16, 16
NW = NC * NS
GW = 128
CHUNK = L
ROWS_PER_W = B // NW
_SPLITS = [(0, GW), (GW, CHUNK - GW)]

MXU_BLK = 5000

_PERM = [(p // 32) * 32 + (p % 2) * 16 + (p % 32) // 2 for p in range(D)]


def _proj_block_kernel(t_ref, w_ref, o_ref):
    o_ref[...] = jnp.dot(t_ref[...], w_ref[...],
                         preferred_element_type=jnp.float32
                         ).astype(jnp.bfloat16)


def _proj_bias_kernel(t_ref, w_ref, b_ref, o_ref):
    o_ref[...] = (jnp.dot(t_ref[...], w_ref[...],
                          preferred_element_type=jnp.float32)
                  + b_ref[...]).astype(jnp.bfloat16)


def _project_meter(table, w):
    return pl.pallas_call(
        _proj_block_kernel,
        grid=(METER_VOCAB // MXU_BLK,),
        in_specs=[pl.BlockSpec((MXU_BLK, D), lambda i: (i, 0)),
                  pl.BlockSpec((D, D), lambda i: (0, 0))],
        out_specs=pl.BlockSpec((MXU_BLK, D), lambda i: (i, 0)),
        out_shape=jax.ShapeDtypeStruct((METER_VOCAB, D), jnp.bfloat16),
    )(table, w)


def _project_small(table, w, bias=None):
    if bias is None:
        return pl.pallas_call(
            _proj_block_kernel,
            out_shape=jax.ShapeDtypeStruct(table.shape, jnp.bfloat16),
        )(table, w)
    return pl.pallas_call(
        _proj_bias_kernel,
        out_shape=jax.ShapeDtypeStruct(table.shape, jnp.bfloat16),
    )(table, w, bias)


_mesh = plsc.VectorSubcoreMesh(core_axis_name="c", subcore_axis_name="s")

_scratch = []
for _slot in range(2):
    _scratch += [pltpu.VMEM((CHUNK,), jnp.int32)] * 3
    _scratch += [pltpu.VMEM((CHUNK, D), jnp.bfloat16)] * 3
    _scratch += [pltpu.VMEM((CHUNK, D), jnp.float32)]
_scratch += [pltpu.SemaphoreType.DMA] * 6


@functools.partial(
    pl.kernel,
    out_type=jax.ShapeDtypeStruct((B, L, D), jnp.float32),
    mesh=_mesh,
    compiler_params=pltpu.CompilerParams(use_tc_tiling_on_sc=False,
                                         needs_layout_passes=False),
    scratch_types=_scratch,
)
def _sc_gather_sum(pm_hbm, plb_hbm, pr_hbm, im_hbm, il_hbm, ir_hbm, out_hbm,
                   im0, il0, ir0, bm0, bl0, br0, o0,
                   im1, il1, ir1, bm1, bl1, br1, o1,
                   semi0, semi1, semg0, semg1, semo0, semo1):
    wid = lax.axis_index("s") * NC + lax.axis_index("c")
    idx_v = ((im0, il0, ir0), (im1, il1, ir1))
    bufs = ((bm0, bl0, br0), (bm1, bl1, br1))
    o_v = (o0, o1)
    semi = (semi0, semi1)
    semg = (semg0, semg1)
    semo = (semo0, semo1)
    idx_hbm = (im_hbm, il_hbm, ir_hbm)
    tables = (pm_hbm, plb_hbm, pr_hbm)
    row_base = wid * ROWS_PER_W

    def load_idx(b, p, is_async):
        for t in range(3):
            src = idx_hbm[t].at[b]
            if is_async:
                pltpu.async_copy(src, idx_v[p][t], semi[p])
            else:
                pltpu.sync_copy(src, idx_v[p][t])

    def wait_idx(p):
        for t in range(3):
            pltpu.make_async_copy(idx_hbm[t].at[0], idx_v[p][t],
                                  semi[p]).wait()

    def fire_gathers(p):
        for t in range(3):
            for (off, n) in _SPLITS:
                pltpu.async_copy(
                    tables[t].at[idx_v[p][t].at[pl.ds(off, n)]],
                    bufs[p][t].at[pl.ds(off, n)],
                    semg[p])

    def drain_gathers(p):
        for t in range(3):
            for (off, n) in _SPLITS:
                pltpu.make_async_copy(
                    tables[t].at[idx_v[p][t].at[pl.ds(off, n)]],
                    bufs[p][t].at[pl.ds(off, n)],
                    semg[p]).wait()

    load_idx(row_base, 0, False)
    fire_gathers(0)
    load_idx(row_base + 1, 1, True)

    @pl.loop(0, ROWS_PER_W, step=2)
    def _pair(ci0):
        for p in range(2):
            q = 1 - p
            ci = ci0 + p
            bm, bl, br = bufs[p]

            @pl.when(ci + 1 < ROWS_PER_W)
            def _():
                wait_idx(q)
                fire_gathers(q)

            drain_gathers(p)

            @pl.when(ci + 2 < ROWS_PER_W)
            def _():
                load_idx(row_base + ci + 2, p, True)

            @pl.when(ci >= 2)
            def _():
                pltpu.make_async_copy(out_hbm.at[0], o_v[p], semo[p]).wait()

            @pl.loop(0, CHUNK, step=2)
            def _row(r):
                for rr in range(2):
                    for j in range(D // 32):
                        sl = (r + rr, pl.ds(j * 32, 32))
                        s = bm[sl] + bl[sl] + br[sl]
                        lo, hi = plsc.unpack(
                            s, format=plsc.PackFormat.INTERLEAVED)
                        o_v[p][r + rr, pl.ds(j * 32, LANES)] = lo
                        o_v[p][r + rr, pl.ds(j * 32 + LANES, LANES)] = hi

            pltpu.async_copy(o_v[p], out_hbm.at[row_base + ci], semo[p])

    pltpu.make_async_copy(out_hbm.at[0], o0, semo0).wait()
    pltpu.make_async_copy(out_hbm.at[0], o1, semo1).wait()


def kernel(meter, length, remainder, meter_table, leng_table, rem_table, W, b):
    perm = jnp.asarray(_PERM, dtype=jnp.int32)
    Wp = W[:, perm]
    bp = b[perm]
    pm = _project_meter(meter_table, Wp[:D])
    plb = _project_small(leng_table, Wp[D:2 * D], bp.reshape(1, D))
    pr = _project_small(rem_table, Wp[2 * D:])
    im = meter.astype(jnp.int32)
    il = length.astype(jnp.int32)
    ir = remainder.astype(jnp.int32)
    return _sc_gather_sum(pm, plb, pr, im, il, ir)
